# Initial kernel scaffold; baseline (speedup 1.0000x reference)
import jax
import jax.numpy as jnp
from jax.experimental import pallas as pl


def kernel(x, W, b, D):
    raise NotImplementedError("write your pallas kernel here")


# speedup vs baseline: 8.1608x; 8.1608x over previous
import functools

import jax
import jax.numpy as jnp
from jax.experimental import pallas as pl
from jax.experimental.pallas import tpu as pltpu

_K = 8
_BLK = 128


def _topk_mask_kernel(d_ref, gn_ref, w_ref, aw_ref, mx_ref):
    p = d_ref[...] + gn_ref[...]
    f = p.shape[1]
    colid = jax.lax.broadcasted_iota(jnp.int32, p.shape, 1)
    mask = jnp.zeros(p.shape, jnp.float32)
    for _ in range(_K):
        m = jnp.max(p, axis=1, keepdims=True)
        is_m = p == m
        first = jnp.min(jnp.where(is_m, colid, f), axis=1, keepdims=True)
        sel = colid == first
        mask = mask + sel.astype(jnp.float32)
        p = jnp.where(sel, -jnp.inf, p)
    aw = mask * w_ref[...]
    aw_ref[...] = aw
    mx_ref[...] = jnp.full((1, 1, 128), jnp.max(jnp.abs(aw)), jnp.float32)


def _matmul_kernel(scale_ref, x_ref, aw_ref, b_ref, y_ref, ymx_ref):
    scale = scale_ref[0, 0]
    wq = jnp.round(scale * aw_ref[...])
    bq = jnp.round(scale * b_ref[...])
    y = jax.lax.dot_general(
        x_ref[...], wq, (((1,), (1,)), ((), ())),
        preferred_element_type=jnp.float32)
    y = jnp.maximum(y + bq, 0.0)
    y_ref[...] = y
    ymx_ref[...] = jnp.full((1, 1, 128), jnp.max(y), jnp.float32)


def _quant_kernel(scale2_ref, y_ref, out_ref):
    out_ref[...] = jnp.round(scale2_ref[0, 0] * y_ref[...])


def kernel(x, W, b, D):
    units, feat = D.shape
    batch = x.shape[0]
    nblk = units // _BLK

    u_rand = jax.random.uniform(jax.random.key(1234), (1, units, feat),
                                minval=0.0, maxval=1.0)
    gn = (-0.1 * jnp.log(-jnp.log(u_rand + 1e-20) + 1e-20))[0]

    aw, mx = pl.pallas_call(
        _topk_mask_kernel,
        grid=(nblk,),
        in_specs=[
            pl.BlockSpec((_BLK, feat), lambda i: (i, 0)),
            pl.BlockSpec((_BLK, feat), lambda i: (i, 0)),
            pl.BlockSpec((_BLK, feat), lambda i: (i, 0)),
        ],
        out_specs=[
            pl.BlockSpec((_BLK, feat), lambda i: (i, 0)),
            pl.BlockSpec((1, 1, 128), lambda i: (i, 0, 0)),
        ],
        out_shape=[
            jax.ShapeDtypeStruct((units, feat), jnp.float32),
            jax.ShapeDtypeStruct((nblk, 1, 128), jnp.float32),
        ],
    )(D, gn, W)

    scale = jnp.reshape(127.5 / jnp.max(mx), (1, 1))
    b2 = jnp.reshape(b, (1, units))

    y, ymx = pl.pallas_call(
        _matmul_kernel,
        grid=(nblk,),
        in_specs=[
            pl.BlockSpec(memory_space=pltpu.SMEM),
            pl.BlockSpec((batch, feat), lambda j: (0, 0)),
            pl.BlockSpec((_BLK, feat), lambda j: (j, 0)),
            pl.BlockSpec((1, _BLK), lambda j: (0, j)),
        ],
        out_specs=[
            pl.BlockSpec((batch, _BLK), lambda j: (0, j)),
            pl.BlockSpec((1, 1, 128), lambda j: (j, 0, 0)),
        ],
        out_shape=[
            jax.ShapeDtypeStruct((batch, units), jnp.float32),
            jax.ShapeDtypeStruct((nblk, 1, 128), jnp.float32),
        ],
    )(scale, x, aw, b2)

    scale2 = jnp.reshape(127.5 / jnp.max(ymx), (1, 1))

    out = pl.pallas_call(
        _quant_kernel,
        in_specs=[
            pl.BlockSpec(memory_space=pltpu.SMEM),
            pl.BlockSpec((batch, units), lambda: (0, 0)),
        ],
        out_specs=pl.BlockSpec((batch, units), lambda: (0, 0)),
        out_shape=jax.ShapeDtypeStruct((batch, units), jnp.float32),
    )(scale2, y)

    return out


# speedup vs baseline: 21.0692x; 2.5817x over previous
import functools

import jax
import jax.numpy as jnp
from jax.experimental import pallas as pl
from jax.experimental.pallas import tpu as pltpu

_K = 8
_BLK = 128


def _topk_mask_kernel(d_ref, gn_ref, w_ref, aw_ref, mx_ref):
    p0 = d_ref[...] + gn_ref[...]
    f = p0.shape[1]

    p = p0
    m = None
    for _ in range(_K):
        m = jnp.max(p, axis=1, keepdims=True)
        p = jnp.where(p == m, -jnp.inf, p)
    ge = (p0 >= m).astype(jnp.float32)
    cnt = jnp.sum(ge, axis=1, keepdims=True)
    exact = jnp.all(cnt == float(_K))

    @pl.when(exact)
    def _():
        aw = ge * w_ref[...]
        aw_ref[...] = aw
        mx_ref[...] = jnp.full((1, 1, 128), jnp.max(jnp.abs(aw)), jnp.float32)

    @pl.when(jnp.logical_not(exact))
    def _():
        colid = jax.lax.broadcasted_iota(jnp.int32, p0.shape, 1)
        mask = jnp.zeros(p0.shape, jnp.float32)
        q = p0
        for _ in range(_K):
            mq = jnp.max(q, axis=1, keepdims=True)
            first = jnp.min(jnp.where(q == mq, colid, f), axis=1,
                            keepdims=True)
            sel = colid == first
            mask = mask + sel.astype(jnp.float32)
            q = jnp.where(sel, -jnp.inf, q)
        aw = mask * w_ref[...]
        aw_ref[...] = aw
        mx_ref[...] = jnp.full((1, 1, 128), jnp.max(jnp.abs(aw)), jnp.float32)


def _matmul_kernel(scale_ref, x_ref, aw_ref, b_ref, y_ref, ymx_ref):
    scale = scale_ref[0, 0]
    wq = jnp.round(scale * aw_ref[...])
    bq = jnp.round(scale * b_ref[...])
    y = jax.lax.dot_general(
        x_ref[...], wq, (((1,), (1,)), ((), ())),
        preferred_element_type=jnp.float32)
    y = jnp.maximum(y + bq, 0.0)
    y_ref[...] = y
    ymx_ref[...] = jnp.full((1, 1, 128), jnp.max(y), jnp.float32)


def _quant_kernel(scale2_ref, y_ref, out_ref):
    out_ref[...] = jnp.round(scale2_ref[0, 0] * y_ref[...])


def kernel(x, W, b, D):
    units, feat = D.shape
    batch = x.shape[0]
    nblk = units // _BLK

    with jax.ensure_compile_time_eval():
        u_rand = jax.random.uniform(jax.random.key(1234), (1, units, feat),
                                    minval=0.0, maxval=1.0)
        gn = (-0.1 * jnp.log(-jnp.log(u_rand + 1e-20) + 1e-20))[0]

    aw, mx = pl.pallas_call(
        _topk_mask_kernel,
        grid=(nblk,),
        in_specs=[
            pl.BlockSpec((_BLK, feat), lambda i: (i, 0)),
            pl.BlockSpec((_BLK, feat), lambda i: (i, 0)),
            pl.BlockSpec((_BLK, feat), lambda i: (i, 0)),
        ],
        out_specs=[
            pl.BlockSpec((_BLK, feat), lambda i: (i, 0)),
            pl.BlockSpec((1, 1, 128), lambda i: (i, 0, 0)),
        ],
        out_shape=[
            jax.ShapeDtypeStruct((units, feat), jnp.float32),
            jax.ShapeDtypeStruct((nblk, 1, 128), jnp.float32),
        ],
    )(D, gn, W)

    scale = jnp.reshape(127.5 / jnp.max(mx), (1, 1))
    b2 = jnp.reshape(b, (1, units))

    y, ymx = pl.pallas_call(
        _matmul_kernel,
        grid=(nblk,),
        in_specs=[
            pl.BlockSpec(memory_space=pltpu.SMEM),
            pl.BlockSpec((batch, feat), lambda j: (0, 0)),
            pl.BlockSpec((_BLK, feat), lambda j: (j, 0)),
            pl.BlockSpec((1, _BLK), lambda j: (0, j)),
        ],
        out_specs=[
            pl.BlockSpec((batch, _BLK), lambda j: (0, j)),
            pl.BlockSpec((1, 1, 128), lambda j: (j, 0, 0)),
        ],
        out_shape=[
            jax.ShapeDtypeStruct((batch, units), jnp.float32),
            jax.ShapeDtypeStruct((nblk, 1, 128), jnp.float32),
        ],
    )(scale, x, aw, b2)

    scale2 = jnp.reshape(127.5 / jnp.max(ymx), (1, 1))

    out = pl.pallas_call(
        _quant_kernel,
        in_specs=[
            pl.BlockSpec(memory_space=pltpu.SMEM),
            pl.BlockSpec((batch, units), lambda: (0, 0)),
        ],
        out_specs=pl.BlockSpec((batch, units), lambda: (0, 0)),
        out_shape=jax.ShapeDtypeStruct((batch, units), jnp.float32),
    )(scale2, y)

    return out
